# baseline (speedup 1.0000x reference)
import jax
import jax.numpy as jnp
from jax.experimental import pallas as pl
from jax.experimental.pallas import tpu as pltpu

_STEPS_PER_CHUNK = 16


def _bcast_body(logs_ref, out_ref, buf_ref, sems):
    buf_ref[...] = jnp.broadcast_to(logs_ref[...], buf_ref.shape)
    n_steps = out_ref.shape[0]
    n_full = n_steps // _STEPS_PER_CHUNK
    for i in range(n_full):
        pltpu.make_async_copy(
            buf_ref,
            out_ref.at[pl.ds(i * _STEPS_PER_CHUNK, _STEPS_PER_CHUNK)],
            sems.at[i],
        ).start()
    rem = n_steps - n_full * _STEPS_PER_CHUNK
    if rem:
        pltpu.make_async_copy(
            buf_ref.at[pl.ds(0, rem)],
            out_ref.at[pl.ds(n_full * _STEPS_PER_CHUNK, rem)],
            sems.at[n_full],
        ).start()
    for i in range(n_full):
        pltpu.make_async_copy(
            buf_ref,
            out_ref.at[pl.ds(i * _STEPS_PER_CHUNK, _STEPS_PER_CHUNK)],
            sems.at[i],
        ).wait()
    if rem:
        pltpu.make_async_copy(
            buf_ref.at[pl.ds(0, rem)],
            out_ref.at[pl.ds(n_full * _STEPS_PER_CHUNK, rem)],
            sems.at[n_full],
        ).wait()


def kernel(hist, logs):
    S_, B_ = hist.shape
    V = logs.shape[0]
    n_chunks = (S_ + 1 + _STEPS_PER_CHUNK - 1) // _STEPS_PER_CHUNK

    logs3d = logs.reshape(1, 1, V)
    return pl.pallas_call(
        _bcast_body,
        in_specs=[pl.BlockSpec(memory_space=pltpu.VMEM)],
        out_specs=pl.BlockSpec(memory_space=pl.ANY),
        out_shape=jax.ShapeDtypeStruct((S_ + 1, B_, V), logs.dtype),
        scratch_shapes=[
            pltpu.VMEM((_STEPS_PER_CHUNK, B_, V), logs.dtype),
            pltpu.SemaphoreType.DMA((n_chunks,)),
        ],
    )(logs3d)
